# baseline (speedup 1.0000x reference)
import functools

import jax
import jax.numpy as jnp
from jax import lax
from jax.experimental import pallas as pl
from jax.experimental.pallas import tpu as pltpu
from jax.experimental.pallas import tpu_sc as plsc

_EMB = 512
_EPS = 1e-08


def _sc_loss_partials(x, label, centers_table):
    B, S, D = x.shape
    info = plsc.get_sparse_core_info()
    nc = info.num_cores
    nw = nc * info.num_subcores
    bpw = B // nw
    GR = 64
    CR = 4
    nch = bpw // CR
    nt = D // 16
    mesh = plsc.VectorSubcoreMesh(core_axis_name="c", subcore_axis_name="s")
    x2 = x.reshape(B, S * D)

    @functools.partial(
        pl.kernel,
        mesh=mesh,
        out_type=(
            jax.ShapeDtypeStruct((B * S * 16,), jnp.float32),
            jax.ShapeDtypeStruct((B * S * 16,), jnp.float32),
            jax.ShapeDtypeStruct((B * 16,), jnp.float32),
        ),
        scratch_types=[
            pltpu.VMEM((GR,), jnp.int32),
            pltpu.VMEM((GR, D), jnp.float32),
            pltpu.VMEM((CR, S * D), jnp.float32),
            pltpu.VMEM((CR, S * D), jnp.float32),
            pltpu.VMEM((bpw * S * 16,), jnp.float32),
            pltpu.VMEM((bpw * S * 16,), jnp.float32),
            pltpu.VMEM((bpw * 16,), jnp.float32),
            pltpu.SemaphoreType.DMA,
            pltpu.SemaphoreType.DMA,
        ],
    )
    def k(x_hbm, lab_hbm, tab_hbm, dout, xout, cout,
          idx_v, crows, xb0, xb1, nums, xns, cns, gsem, xsem):
        wid = lax.axis_index("s") * nc + lax.axis_index("c")
        base = wid * bpw

        def gather_group(g):
            pltpu.sync_copy(lab_hbm.at[pl.ds(base + g * GR, GR)], idx_v)
            pltpu.async_copy(tab_hbm.at[idx_v], crows, gsem).wait()

        gather_group(0)
        pltpu.async_copy(x_hbm.at[pl.ds(base, CR)], xb0, xsem)

        def compute_chunk(ch_idx, xb):
            def bbody(b, carry):
                row = ch_idx * CR + b
                grow = lax.rem(row, GR)
                cvs = [crows[grow, pl.ds(t * 16, 16)] for t in range(nt)]
                cacc = cvs[0] * cvs[0]
                for t in range(1, nt):
                    cacc = cacc + cvs[t] * cvs[t]
                cns[pl.ds(row * 16, 16)] = cacc
                for s in range(S):
                    xvs = [xb[b, pl.ds(s * D + t * 16, 16)] for t in range(nt)]
                    accd = xvs[0] * cvs[0]
                    accn = xvs[0] * xvs[0]
                    for t in range(1, nt):
                        accd = accd + xvs[t] * cvs[t]
                        accn = accn + xvs[t] * xvs[t]
                    nums[pl.ds((row * S + s) * 16, 16)] = accd
                    xns[pl.ds((row * S + s) * 16, 16)] = accn
                return carry

            lax.fori_loop(0, CR, bbody, 0)

        def jbody(j, carry):
            ch0 = 2 * j

            @pl.when(ch0 == GR // CR)
            def _regather():
                gather_group(1)

            pltpu.make_async_copy(
                x_hbm.at[pl.ds(base, CR)], xb0, xsem).wait()
            pltpu.async_copy(
                x_hbm.at[pl.ds(base + (ch0 + 1) * CR, CR)], xb1, xsem)
            compute_chunk(ch0, xb0)
            pltpu.make_async_copy(
                x_hbm.at[pl.ds(base, CR)], xb1, xsem).wait()
            nxt = jnp.minimum(ch0 + 2, nch - 1)
            pltpu.async_copy(x_hbm.at[pl.ds(base + nxt * CR, CR)], xb0, xsem)
            compute_chunk(ch0 + 1, xb1)
            return carry

        lax.fori_loop(0, nch // 2, jbody, 0)
        pltpu.make_async_copy(x_hbm.at[pl.ds(base, CR)], xb0, xsem).wait()
        pltpu.sync_copy(nums, dout.at[pl.ds(base * S * 16, bpw * S * 16)])
        pltpu.sync_copy(xns, xout.at[pl.ds(base * S * 16, bpw * S * 16)])
        pltpu.sync_copy(cns, cout.at[pl.ds(base * 16, bpw * 16)])

    return k(x2, label, centers_table)


def _tail_body(d_ref, n_ref, c_ref, out_ref):
    lanes = d_ref.shape[1]
    grp = jnp.where(
        lax.broadcasted_iota(jnp.int32, (lanes, lanes // 16), 0) // 16
        == lax.broadcasted_iota(jnp.int32, (lanes, lanes // 16), 1),
        1.0, 0.0)
    num = jax.lax.dot(d_ref[...], grp)
    xn2 = jax.lax.dot(n_ref[...], grp)
    cn2 = jax.lax.dot(c_ref[...], jnp.ones((16, 1), jnp.float32))
    q = num * lax.rsqrt(jnp.maximum(xn2 * cn2, _EPS * _EPS))
    out_ref[0, 0] = jnp.sum(q)


def kernel(x, label, centers_table):
    B, S, D = x.shape
    dots_p, xn_p, cn_p = _sc_loss_partials(x, label, centers_table)
    total = pl.pallas_call(
        _tail_body,
        out_specs=pl.BlockSpec(memory_space=pltpu.SMEM),
        out_shape=jax.ShapeDtypeStruct((1, 1), jnp.float32),
    )(dots_p.reshape(B, S * 16), xn_p.reshape(B, S * 16),
      cn_p.reshape(B, 16))
    return total[0, 0] / (B * S)


# speedup vs baseline: 1.0030x; 1.0030x over previous
import functools

import jax
import jax.numpy as jnp
from jax import lax
from jax.experimental import pallas as pl
from jax.experimental.pallas import tpu as pltpu
from jax.experimental.pallas import tpu_sc as plsc

_EMB = 512
_EPS = 1e-08


def _sc_loss_partials(x, label, centers_table):
    B, S, D = x.shape
    info = plsc.get_sparse_core_info()
    nc = info.num_cores
    nw = nc * info.num_subcores
    bpw = B // nw
    GR = 64
    CR = 4
    nch = bpw // CR
    nt = D // 16
    mesh = plsc.VectorSubcoreMesh(core_axis_name="c", subcore_axis_name="s")
    x2 = x.reshape(B, S * D)

    @functools.partial(
        pl.kernel,
        mesh=mesh,
        out_type=(
            jax.ShapeDtypeStruct((B * S * 16,), jnp.float32),
            jax.ShapeDtypeStruct((B * S * 16,), jnp.float32),
            jax.ShapeDtypeStruct((B * 16,), jnp.float32),
        ),
        scratch_types=[
            pltpu.VMEM((GR,), jnp.int32),
            pltpu.VMEM((GR, D), jnp.float32),
            pltpu.VMEM((CR, S * D), jnp.float32),
            pltpu.VMEM((CR, S * D), jnp.float32),
            pltpu.VMEM((bpw * S * 16,), jnp.float32),
            pltpu.VMEM((bpw * S * 16,), jnp.float32),
            pltpu.VMEM((bpw * 16,), jnp.float32),
            pltpu.SemaphoreType.DMA,
            pltpu.SemaphoreType.DMA,
        ],
    )
    def k(x_hbm, lab_hbm, tab_hbm, dout, xout, cout,
          idx_v, crows, xb0, xb1, nums, xns, cns, gsem, xsem):
        wid = lax.axis_index("s") * nc + lax.axis_index("c")
        base = wid * bpw

        def gather_group(g):
            pltpu.sync_copy(lab_hbm.at[pl.ds(base + g * GR, GR)], idx_v)
            pltpu.async_copy(tab_hbm.at[idx_v], crows, gsem).wait()

        gather_group(0)
        pltpu.async_copy(x_hbm.at[pl.ds(base, CR)], xb0, xsem)

        def compute_chunk(ch_idx, xb):
            def tree(vals):
                while len(vals) > 1:
                    vals = [a + b for a, b in zip(vals[::2], vals[1::2])]
                return vals[0]

            def bbody(b, carry):
                row = ch_idx * CR + b
                grow = lax.rem(row, GR)
                cvs = [crows[grow, pl.ds(t * 16, 16)] for t in range(nt)]
                cns[pl.ds(row * 16, 16)] = tree([cv * cv for cv in cvs])
                for s in range(S):
                    xvs = [xb[b, pl.ds(s * D + t * 16, 16)] for t in range(nt)]
                    nums[pl.ds((row * S + s) * 16, 16)] = tree(
                        [xv * cv for xv, cv in zip(xvs, cvs)])
                    xns[pl.ds((row * S + s) * 16, 16)] = tree(
                        [xv * xv for xv in xvs])
                return carry

            lax.fori_loop(0, CR, bbody, 0)

        def jbody(j, carry):
            ch0 = 2 * j

            @pl.when(ch0 == GR // CR)
            def _regather():
                gather_group(1)

            pltpu.make_async_copy(
                x_hbm.at[pl.ds(base, CR)], xb0, xsem).wait()
            pltpu.async_copy(
                x_hbm.at[pl.ds(base + (ch0 + 1) * CR, CR)], xb1, xsem)
            compute_chunk(ch0, xb0)
            pltpu.make_async_copy(
                x_hbm.at[pl.ds(base, CR)], xb1, xsem).wait()
            nxt = jnp.minimum(ch0 + 2, nch - 1)
            pltpu.async_copy(x_hbm.at[pl.ds(base + nxt * CR, CR)], xb0, xsem)
            compute_chunk(ch0 + 1, xb1)
            return carry

        lax.fori_loop(0, nch // 2, jbody, 0)
        pltpu.make_async_copy(x_hbm.at[pl.ds(base, CR)], xb0, xsem).wait()
        pltpu.sync_copy(nums, dout.at[pl.ds(base * S * 16, bpw * S * 16)])
        pltpu.sync_copy(xns, xout.at[pl.ds(base * S * 16, bpw * S * 16)])
        pltpu.sync_copy(cns, cout.at[pl.ds(base * 16, bpw * 16)])

    return k(x2, label, centers_table)


def _tail_body(d_ref, n_ref, c_ref, out_ref):
    lanes = d_ref.shape[1]
    grp = jnp.where(
        lax.broadcasted_iota(jnp.int32, (lanes, lanes // 16), 0) // 16
        == lax.broadcasted_iota(jnp.int32, (lanes, lanes // 16), 1),
        1.0, 0.0)
    num = jax.lax.dot(d_ref[...], grp)
    xn2 = jax.lax.dot(n_ref[...], grp)
    cn2 = jax.lax.dot(c_ref[...], jnp.ones((16, 1), jnp.float32))
    q = num * lax.rsqrt(jnp.maximum(xn2 * cn2, _EPS * _EPS))
    out_ref[0, 0] = jnp.sum(q)


def kernel(x, label, centers_table):
    B, S, D = x.shape
    dots_p, xn_p, cn_p = _sc_loss_partials(x, label, centers_table)
    total = pl.pallas_call(
        _tail_body,
        out_specs=pl.BlockSpec(memory_space=pltpu.SMEM),
        out_shape=jax.ShapeDtypeStruct((1, 1), jnp.float32),
    )(dots_p.reshape(B, S * 16), xn_p.reshape(B, S * 16),
      cn_p.reshape(B, 16))
    return total[0, 0] / (B * S)


# speedup vs baseline: 2.4304x; 2.4230x over previous
import functools

import jax
import jax.numpy as jnp
from jax import lax
from jax.experimental import pallas as pl
from jax.experimental.pallas import tpu as pltpu
from jax.experimental.pallas import tpu_sc as plsc

_EMB = 512
_EPS = 1e-08


def _gather_centers(centers_table, label, off, cb):
    info = plsc.get_sparse_core_info()
    nc = info.num_cores
    nw = nc * info.num_subcores
    b_per_w = cb // nw
    mesh = plsc.VectorSubcoreMesh(core_axis_name="c", subcore_axis_name="s")

    @functools.partial(
        pl.kernel,
        mesh=mesh,
        out_type=jax.ShapeDtypeStruct((cb, _EMB), jnp.float32),
        scratch_types=[
            pltpu.VMEM((b_per_w,), jnp.int32),
            pltpu.VMEM((b_per_w, _EMB), jnp.float32),
            pltpu.SemaphoreType.DMA,
        ],
    )
    def gather_k(table_hbm, idx_hbm, out_hbm, idx_v, rows_v, sem):
        wid = lax.axis_index("s") * nc + lax.axis_index("c")
        base = wid * b_per_w
        pltpu.sync_copy(idx_hbm.at[pl.ds(off + base, b_per_w)], idx_v)
        pltpu.async_copy(table_hbm.at[idx_v], rows_v, sem).wait()
        pltpu.sync_copy(rows_v, out_hbm.at[pl.ds(base, b_per_w)])

    return gather_k(centers_table, label)


def _loss_body(x_ref, c_ref, acc_ref):
    x = x_ref[...]
    c = c_ref[...]
    dots = jnp.sum(x * c[:, None, :], axis=-1)
    xn2 = jnp.sum(x * x, axis=-1)
    cn2 = jnp.sum(c * c, axis=-1)
    q = dots * lax.rsqrt(jnp.maximum(xn2 * cn2[:, None], _EPS * _EPS))
    part = jnp.sum(q)

    @pl.when(pl.program_id(0) == 0)
    def _init():
        acc_ref[0, 0] = 0.0

    acc_ref[0, 0] += part


def _loss_chunk(x, centers, chunk, cb, bb):
    s, d = x.shape[1], x.shape[2]
    return pl.pallas_call(
        _loss_body,
        grid=(cb // bb,),
        in_specs=[
            pl.BlockSpec((bb, s, d), lambda i: (i, 0, 0)),
            pl.BlockSpec((bb, d), lambda i: (i, 0)),
        ],
        out_specs=pl.BlockSpec(memory_space=pltpu.SMEM),
        out_shape=jax.ShapeDtypeStruct((1, 1), jnp.float32),
    )(x, centers)


def kernel(x, label, centers_table):
    B, S, D = x.shape
    BB = 1024
    centers = _gather_centers(centers_table, label, 0, B)
    total = _loss_chunk(x, centers, 0, B, BB)[0, 0]
    return total / (B * S)
